# initial kernel scaffold (unmeasured)
import jax
import jax.numpy as jnp
from jax import lax
from jax.experimental import pallas as pl
from jax.experimental.pallas import tpu as pltpu

N_DEV = 4
M = 4096
KS = 1024
N = 8192
MC = M // N_DEV
NT = 1024
N_TILES = N // NT


def kernel(x, w_mat):
    def body(x_hbm, w_hbm, out_hbm, wbf, stage, xcbf, buf_a, buf_b,
             outstage, local_sem, send_sem, recv_sem):
        i = lax.axis_index("i")
        right = lax.rem(i + 1, N_DEV)
        left = lax.rem(i + N_DEV - 1, N_DEV)

        for nt in range(N_TILES):
            cp = pltpu.make_async_copy(
                w_hbm.at[:, pl.ds(nt * NT, NT)], stage, local_sem)
            cp.start()
            cp.wait()
            wbf[:, nt * NT:(nt + 1) * NT] = stage[...].astype(jnp.bfloat16)

        bar = pltpu.get_barrier_semaphore()

        def barrier():
            for nbr in (left, right):
                pl.semaphore_signal(
                    bar, inc=1, device_id=(nbr,),
                    device_id_type=pl.DeviceIdType.MESH)
            pl.semaphore_wait(bar, 2)

        def load_x_chunk(c):
            cp = pltpu.make_async_copy(
                x_hbm.at[pl.ds(c * MC, MC), :], stage, local_sem)
            cp.start()
            cp.wait()
            xcbf[...] = stage[...].astype(jnp.bfloat16)

        def gemm_chunk(c, dst, add=None):
            load_x_chunk(c)
            for nt in range(N_TILES):
                sl = slice(nt * NT, (nt + 1) * NT)
                t = jnp.dot(xcbf[...], wbf[:, sl],
                            preferred_element_type=jnp.float32)
                if add is not None:
                    t = t + add[:, sl].astype(jnp.float32)
                dst[:, sl] = t.astype(jnp.bfloat16)

        def ring_step(src, dst):
            rdma = pltpu.make_async_remote_copy(
                src_ref=src, dst_ref=dst,
                send_sem=send_sem, recv_sem=recv_sem,
                device_id=(right,), device_id_type=pl.DeviceIdType.MESH)
            rdma.start()
            rdma.wait()

        def store_out(buf, c):
            for nt in range(N_TILES):
                sl = slice(nt * NT, (nt + 1) * NT)
                outstage[...] = buf[:, sl].astype(jnp.float32)
                cp = pltpu.make_async_copy(
                    outstage,
                    out_hbm.at[pl.ds(c * MC, MC), pl.ds(nt * NT, NT)],
                    local_sem)
                cp.start()
                cp.wait()

        barrier()

        gemm_chunk(i, buf_a)
        barrier()
        ring_step(buf_a, buf_b)

        gemm_chunk(lax.rem(i + 3, N_DEV), buf_b, add=buf_b)
        barrier()
        ring_step(buf_b, buf_a)

        gemm_chunk(lax.rem(i + 2, N_DEV), buf_a, add=buf_a)
        barrier()
        ring_step(buf_a, buf_b)

        o = lax.rem(i + 1, N_DEV)
        load_x_chunk(o)
        for nt in range(N_TILES):
            sl = slice(nt * NT, (nt + 1) * NT)
            t = jnp.dot(xcbf[...], wbf[:, sl],
                        preferred_element_type=jnp.float32)
            t = t + buf_b[:, sl].astype(jnp.float32)
            t = jnp.maximum(t, 0.0)
            buf_a[:, sl] = t.astype(jnp.bfloat16)
            outstage[...] = t
            cp = pltpu.make_async_copy(
                outstage,
                out_hbm.at[pl.ds(o * MC, MC), pl.ds(nt * NT, NT)],
                local_sem)
            cp.start()
            cp.wait()

        barrier()
        ring_step(buf_a, buf_b)
        store_out(buf_b, i)

        barrier()
        ring_step(buf_b, buf_a)
        store_out(buf_a, lax.rem(i + 3, N_DEV))

        barrier()
        ring_step(buf_a, buf_b)
        store_out(buf_b, lax.rem(i + 2, N_DEV))

        barrier()

    return pl.pallas_call(
        body,
        out_shape=jax.ShapeDtypeStruct((M, N), jnp.float32),
        in_specs=[
            pl.BlockSpec(memory_space=pltpu.ANY),
            pl.BlockSpec(memory_space=pltpu.ANY),
        ],
        out_specs=pl.BlockSpec(memory_space=pltpu.ANY),
        scratch_shapes=[
            pltpu.VMEM((KS, N), jnp.bfloat16),
            pltpu.VMEM((MC, NT), jnp.float32),
            pltpu.VMEM((MC, KS), jnp.bfloat16),
            pltpu.VMEM((MC, N), jnp.bfloat16),
            pltpu.VMEM((MC, N), jnp.bfloat16),
            pltpu.VMEM((MC, NT), jnp.float32),
            pltpu.SemaphoreType.DMA,
            pltpu.SemaphoreType.DMA,
            pltpu.SemaphoreType.DMA,
        ],
        compiler_params=pltpu.CompilerParams(collective_id=0),
    )(x, w_mat)


# baseline (device time: 1411257 ns/iter reference)
import jax
import jax.numpy as jnp
from jax import lax
from jax.experimental import pallas as pl
from jax.experimental.pallas import tpu as pltpu

N_DEV = 4
M = 4096
KS = 1024
N = 8192
MC = M // N_DEV
NT = 1024
N_TILES = N // NT


def kernel(x, w_mat):
    def body(x_hbm, w_hbm, out_hbm, stage, xcbf, wtbf, buf_a, buf_b,
             outstage, local_sem, send_sem, recv_sem):
        i = lax.axis_index("i")
        right = lax.rem(i + 1, N_DEV)
        left = lax.rem(i + N_DEV - 1, N_DEV)

        bar = pltpu.get_barrier_semaphore()

        def barrier():
            for nbr in (left, right):
                pl.semaphore_signal(
                    bar, inc=1, device_id=(nbr,),
                    device_id_type=pl.DeviceIdType.MESH)
            pl.semaphore_wait(bar, 2)

        def load_x_chunk(c):
            cp = pltpu.make_async_copy(
                x_hbm.at[pl.ds(c * MC, MC), :], stage, local_sem)
            cp.start()
            cp.wait()
            xcbf[...] = stage[...].astype(jnp.bfloat16)

        def load_w_tile(nt):
            cp = pltpu.make_async_copy(
                w_hbm.at[:, pl.ds(nt * NT, NT)], stage, local_sem)
            cp.start()
            cp.wait()
            wtbf[...] = stage[...].astype(jnp.bfloat16)

        def gemm_chunk(c, dst, add=None):
            load_x_chunk(c)

            def tile(nt, _):
                sl = pl.ds(nt * NT, NT)
                load_w_tile(nt)
                t = jnp.dot(xcbf[...], wtbf[...],
                            preferred_element_type=jnp.float32)
                if add is not None:
                    t = t + add[:, sl].astype(jnp.float32)
                dst[:, sl] = t.astype(jnp.bfloat16)
                return 0

            lax.fori_loop(0, N_TILES, tile, 0)

        def ring_step(src, dst):
            rdma = pltpu.make_async_remote_copy(
                src_ref=src, dst_ref=dst,
                send_sem=send_sem, recv_sem=recv_sem,
                device_id=(right,), device_id_type=pl.DeviceIdType.MESH)
            rdma.start()
            rdma.wait()

        def store_out(buf, c):
            def tile(nt, _):
                outstage[...] = buf[:, pl.ds(nt * NT, NT)].astype(jnp.float32)
                cp = pltpu.make_async_copy(
                    outstage,
                    out_hbm.at[pl.ds(c * MC, MC), pl.ds(nt * NT, NT)],
                    local_sem)
                cp.start()
                cp.wait()
                return 0

            lax.fori_loop(0, N_TILES, tile, 0)

        barrier()

        gemm_chunk(i, buf_a)
        barrier()
        ring_step(buf_a, buf_b)

        gemm_chunk(lax.rem(i + 3, N_DEV), buf_b, add=buf_b)
        barrier()
        ring_step(buf_b, buf_a)

        gemm_chunk(lax.rem(i + 2, N_DEV), buf_a, add=buf_a)
        barrier()
        ring_step(buf_a, buf_b)

        o = lax.rem(i + 1, N_DEV)
        load_x_chunk(o)

        def final_tile(nt, _):
            sl = pl.ds(nt * NT, NT)
            load_w_tile(nt)
            t = jnp.dot(xcbf[...], wtbf[...],
                        preferred_element_type=jnp.float32)
            t = t + buf_b[:, sl].astype(jnp.float32)
            t = jnp.maximum(t, 0.0)
            buf_a[:, sl] = t.astype(jnp.bfloat16)
            outstage[...] = t
            cp = pltpu.make_async_copy(
                outstage,
                out_hbm.at[pl.ds(o * MC, MC), sl],
                local_sem)
            cp.start()
            cp.wait()
            return 0

        lax.fori_loop(0, N_TILES, final_tile, 0)

        barrier()
        ring_step(buf_a, buf_b)
        store_out(buf_b, i)

        barrier()
        ring_step(buf_b, buf_a)
        store_out(buf_a, lax.rem(i + 3, N_DEV))

        barrier()
        ring_step(buf_a, buf_b)
        store_out(buf_b, lax.rem(i + 2, N_DEV))

        barrier()

    return pl.pallas_call(
        body,
        out_shape=jax.ShapeDtypeStruct((M, N), jnp.float32),
        in_specs=[
            pl.BlockSpec(memory_space=pl.ANY),
            pl.BlockSpec(memory_space=pl.ANY),
        ],
        out_specs=pl.BlockSpec(memory_space=pl.ANY),
        scratch_shapes=[
            pltpu.VMEM((MC, NT), jnp.float32),
            pltpu.VMEM((MC, KS), jnp.bfloat16),
            pltpu.VMEM((KS, NT), jnp.bfloat16),
            pltpu.VMEM((MC, N), jnp.bfloat16),
            pltpu.VMEM((MC, N), jnp.bfloat16),
            pltpu.VMEM((MC, NT), jnp.float32),
            pltpu.SemaphoreType.DMA,
            pltpu.SemaphoreType.DMA,
            pltpu.SemaphoreType.DMA,
        ],
        compiler_params=pltpu.CompilerParams(
            collective_id=0,
            vmem_limit_bytes=64 * 1024 * 1024,
        ),
    )(x, w_mat)


# device time: 881308 ns/iter; 1.6013x vs baseline; 1.6013x over previous
import jax
import jax.numpy as jnp
from jax import lax
from jax.experimental import pallas as pl
from jax.experimental.pallas import tpu as pltpu

N_DEV = 4
M = 4096
KS = 1024
N = 8192
NH = N // 2
MC = M // N_DEV
NT = 1024
H_TILES = NH // NT


def kernel(x, w_mat):
    def body(x_hbm, w_hbm, out_hbm, stage, xcbf, wtbf, buf_a, buf_b,
             outstage, local_sem, send_r, recv_r, send_l, recv_l):
        i = lax.axis_index("i")
        right = lax.rem(i + 1, N_DEV)
        left = lax.rem(i + N_DEV - 1, N_DEV)

        bar = pltpu.get_barrier_semaphore()

        def barrier():
            for nbr in (left, right):
                pl.semaphore_signal(
                    bar, inc=1, device_id=(nbr,),
                    device_id_type=pl.DeviceIdType.MESH)
            pl.semaphore_wait(bar, 2)

        def load_x_chunk(c):
            cp = pltpu.make_async_copy(
                x_hbm.at[pl.ds(c * MC, MC), :], stage, local_sem)
            cp.start()
            cp.wait()
            xcbf[...] = stage[...].astype(jnp.bfloat16)

        def load_w_tile(col):
            cp = pltpu.make_async_copy(
                w_hbm.at[:, pl.ds(col, NT)], stage, local_sem)
            cp.start()
            cp.wait()
            wtbf[...] = stage[...].astype(jnp.bfloat16)

        def gemm_half(c, dst, col0, add=None):
            load_x_chunk(c)

            def tile(nt, _):
                col = col0 + nt * NT
                sl = pl.ds(col, NT)
                load_w_tile(col)
                t = jnp.dot(xcbf[...], wtbf[...],
                            preferred_element_type=jnp.float32)
                if add is not None:
                    t = t + add[:, sl].astype(jnp.float32)
                dst[:, sl] = t.astype(jnp.bfloat16)
                return 0

            lax.fori_loop(0, H_TILES, tile, 0)

        def ring_step(src, dst):
            r = pltpu.make_async_remote_copy(
                src_ref=src.at[:, pl.ds(0, NH)],
                dst_ref=dst.at[:, pl.ds(0, NH)],
                send_sem=send_r, recv_sem=recv_r,
                device_id=(right,), device_id_type=pl.DeviceIdType.MESH)
            l = pltpu.make_async_remote_copy(
                src_ref=src.at[:, pl.ds(NH, NH)],
                dst_ref=dst.at[:, pl.ds(NH, NH)],
                send_sem=send_l, recv_sem=recv_l,
                device_id=(left,), device_id_type=pl.DeviceIdType.MESH)
            r.start()
            l.start()
            r.wait()
            l.wait()

        def final_half(o, col0):
            load_x_chunk(o)

            def tile(nt, _):
                col = col0 + nt * NT
                sl = pl.ds(col, NT)
                load_w_tile(col)
                t = jnp.dot(xcbf[...], wtbf[...],
                            preferred_element_type=jnp.float32)
                t = t + buf_b[:, sl].astype(jnp.float32)
                t = jnp.maximum(t, 0.0)
                buf_a[:, sl] = t.astype(jnp.bfloat16)
                outstage[...] = t
                cp = pltpu.make_async_copy(
                    outstage, out_hbm.at[pl.ds(o * MC, MC), sl], local_sem)
                cp.start()
                cp.wait()
                return 0

            lax.fori_loop(0, H_TILES, tile, 0)

        def store_half(buf, c, col0):
            def tile(nt, _):
                sl = pl.ds(col0 + nt * NT, NT)
                outstage[...] = buf[:, sl].astype(jnp.float32)
                cp = pltpu.make_async_copy(
                    outstage, out_hbm.at[pl.ds(c * MC, MC), sl], local_sem)
                cp.start()
                cp.wait()
                return 0

            lax.fori_loop(0, H_TILES, tile, 0)

        barrier()

        gemm_half(i, buf_a, 0)
        gemm_half(i, buf_a, NH)
        barrier()
        ring_step(buf_a, buf_b)

        gemm_half(lax.rem(i + 3, N_DEV), buf_b, 0, add=buf_b)
        gemm_half(lax.rem(i + 1, N_DEV), buf_b, NH, add=buf_b)
        barrier()
        ring_step(buf_b, buf_a)

        gemm_half(lax.rem(i + 2, N_DEV), buf_a, 0, add=buf_a)
        gemm_half(lax.rem(i + 2, N_DEV), buf_a, NH, add=buf_a)
        barrier()
        ring_step(buf_a, buf_b)

        final_half(lax.rem(i + 1, N_DEV), 0)
        final_half(lax.rem(i + 3, N_DEV), NH)

        barrier()
        ring_step(buf_a, buf_b)
        store_half(buf_b, i, 0)
        store_half(buf_b, i, NH)

        barrier()
        ring_step(buf_b, buf_a)
        store_half(buf_a, lax.rem(i + 3, N_DEV), 0)
        store_half(buf_a, lax.rem(i + 1, N_DEV), NH)

        barrier()
        ring_step(buf_a, buf_b)
        store_half(buf_b, lax.rem(i + 2, N_DEV), 0)
        store_half(buf_b, lax.rem(i + 2, N_DEV), NH)

        barrier()

    return pl.pallas_call(
        body,
        out_shape=jax.ShapeDtypeStruct((M, N), jnp.float32),
        in_specs=[
            pl.BlockSpec(memory_space=pl.ANY),
            pl.BlockSpec(memory_space=pl.ANY),
        ],
        out_specs=pl.BlockSpec(memory_space=pl.ANY),
        scratch_shapes=[
            pltpu.VMEM((MC, NT), jnp.float32),
            pltpu.VMEM((MC, KS), jnp.bfloat16),
            pltpu.VMEM((KS, NT), jnp.bfloat16),
            pltpu.VMEM((MC, N), jnp.bfloat16),
            pltpu.VMEM((MC, N), jnp.bfloat16),
            pltpu.VMEM((MC, NT), jnp.float32),
            pltpu.SemaphoreType.DMA,
            pltpu.SemaphoreType.DMA,
            pltpu.SemaphoreType.DMA,
            pltpu.SemaphoreType.DMA,
            pltpu.SemaphoreType.DMA,
        ],
        compiler_params=pltpu.CompilerParams(
            collective_id=0,
            vmem_limit_bytes=64 * 1024 * 1024,
        ),
    )(x, w_mat)


# device time: 827297 ns/iter; 1.7059x vs baseline; 1.0653x over previous
import jax
import jax.numpy as jnp
from jax import lax
from jax.experimental import pallas as pl
from jax.experimental.pallas import tpu as pltpu

N_DEV = 4
M = 4096
KS = 1024
N = 8192
NH = N // 2
MC = M // N_DEV
NT = 1024
H_TILES = NH // NT


def kernel(x, w_mat):
    def body(x_hbm, w_hbm, out_hbm, stage, xcbf, wtbf, buf_a, buf_b,
             outstage, local_sem, send_r, recv_r, send_l, recv_l):
        i = lax.axis_index("i")
        right = lax.rem(i + 1, N_DEV)
        left = lax.rem(i + N_DEV - 1, N_DEV)

        bar = pltpu.get_barrier_semaphore()

        def barrier():
            for nbr in (left, right):
                pl.semaphore_signal(
                    bar, inc=1, device_id=(nbr,),
                    device_id_type=pl.DeviceIdType.MESH)
            pl.semaphore_wait(bar, 2)

        def load_x_chunk(c):
            cp = pltpu.make_async_copy(
                x_hbm.at[pl.ds(c * MC, MC), :], stage, local_sem)
            cp.start()
            cp.wait()
            xcbf[...] = stage[...].astype(jnp.bfloat16)

        def load_w_tile(col):
            cp = pltpu.make_async_copy(
                w_hbm.at[:, pl.ds(col, NT)], stage, local_sem)
            cp.start()
            cp.wait()
            wtbf[...] = stage[...].astype(jnp.bfloat16)

        def gemm_half(c, dst, col0, add=None):
            load_x_chunk(c)

            def tile(nt, _):
                col = col0 + nt * NT
                sl = pl.ds(col, NT)
                load_w_tile(col)
                t = jnp.dot(xcbf[...], wtbf[...],
                            preferred_element_type=jnp.float32)
                if add is not None:
                    t = t + add[:, sl].astype(jnp.float32)
                dst[:, sl] = t.astype(jnp.bfloat16)
                return 0

            lax.fori_loop(0, H_TILES, tile, 0)

        def ring_start(src, dst):
            r = pltpu.make_async_remote_copy(
                src_ref=src.at[:, pl.ds(0, NH)],
                dst_ref=dst.at[:, pl.ds(0, NH)],
                send_sem=send_r, recv_sem=recv_r,
                device_id=(right,), device_id_type=pl.DeviceIdType.MESH)
            l = pltpu.make_async_remote_copy(
                src_ref=src.at[:, pl.ds(NH, NH)],
                dst_ref=dst.at[:, pl.ds(NH, NH)],
                send_sem=send_l, recv_sem=recv_l,
                device_id=(left,), device_id_type=pl.DeviceIdType.MESH)
            r.start()
            l.start()
            return r, l

        def ring_wait(rl):
            r, l = rl
            r.wait()
            l.wait()

        def ring_step(src, dst):
            ring_wait(ring_start(src, dst))

        def gemm_full(c, dst, add=None):
            load_x_chunk(c)

            def tile(nt, _):
                sl = pl.ds(nt * NT, NT)
                load_w_tile(nt * NT)
                t = jnp.dot(xcbf[...], wtbf[...],
                            preferred_element_type=jnp.float32)
                if add is not None:
                    t = t + add[:, sl].astype(jnp.float32)
                dst[:, sl] = t.astype(jnp.bfloat16)
                return 0

            lax.fori_loop(0, 2 * H_TILES, tile, 0)

        def final_half(o, col0):
            load_x_chunk(o)

            def tile(nt, _):
                sl = pl.ds(col0 + nt * NT, NT)
                load_w_tile(col0 + nt * NT)
                t = jnp.dot(xcbf[...], wtbf[...],
                            preferred_element_type=jnp.float32)
                t = t + buf_b[:, sl].astype(jnp.float32)
                t = jnp.maximum(t, 0.0)
                buf_a[:, sl] = t.astype(jnp.bfloat16)
                return 0

            lax.fori_loop(0, H_TILES, tile, 0)

        def store_half(buf, c, col0):
            def tile(nt, _):
                sl = pl.ds(col0 + nt * NT, NT)
                outstage[...] = buf[:, sl].astype(jnp.float32)
                cp = pltpu.make_async_copy(
                    outstage, out_hbm.at[pl.ds(c * MC, MC), sl], local_sem)
                cp.start()
                cp.wait()
                return 0

            lax.fori_loop(0, H_TILES, tile, 0)

        barrier()

        gemm_full(i, buf_a)
        barrier()
        ring_step(buf_a, buf_b)

        gemm_half(lax.rem(i + 3, N_DEV), buf_b, 0, add=buf_b)
        gemm_half(lax.rem(i + 1, N_DEV), buf_b, NH, add=buf_b)
        barrier()
        ring_step(buf_b, buf_a)

        gemm_full(lax.rem(i + 2, N_DEV), buf_a, add=buf_a)
        barrier()
        ring_step(buf_a, buf_b)

        final_half(lax.rem(i + 1, N_DEV), 0)
        final_half(lax.rem(i + 3, N_DEV), NH)

        barrier()
        ag0 = ring_start(buf_a, buf_b)
        store_half(buf_a, lax.rem(i + 1, N_DEV), 0)
        store_half(buf_a, lax.rem(i + 3, N_DEV), NH)
        ring_wait(ag0)

        barrier()
        ag1 = ring_start(buf_b, buf_a)
        store_half(buf_b, i, 0)
        store_half(buf_b, i, NH)
        ring_wait(ag1)

        barrier()
        ag2 = ring_start(buf_a, buf_b)
        store_half(buf_a, lax.rem(i + 3, N_DEV), 0)
        store_half(buf_a, lax.rem(i + 1, N_DEV), NH)
        ring_wait(ag2)

        store_half(buf_b, lax.rem(i + 2, N_DEV), 0)
        store_half(buf_b, lax.rem(i + 2, N_DEV), NH)

        barrier()

    return pl.pallas_call(
        body,
        out_shape=jax.ShapeDtypeStruct((M, N), jnp.float32),
        in_specs=[
            pl.BlockSpec(memory_space=pl.ANY),
            pl.BlockSpec(memory_space=pl.ANY),
        ],
        out_specs=pl.BlockSpec(memory_space=pl.ANY),
        scratch_shapes=[
            pltpu.VMEM((MC, NT), jnp.float32),
            pltpu.VMEM((MC, KS), jnp.bfloat16),
            pltpu.VMEM((KS, NT), jnp.bfloat16),
            pltpu.VMEM((MC, N), jnp.bfloat16),
            pltpu.VMEM((MC, N), jnp.bfloat16),
            pltpu.VMEM((MC, NT), jnp.float32),
            pltpu.SemaphoreType.DMA,
            pltpu.SemaphoreType.DMA,
            pltpu.SemaphoreType.DMA,
            pltpu.SemaphoreType.DMA,
            pltpu.SemaphoreType.DMA,
        ],
        compiler_params=pltpu.CompilerParams(
            collective_id=0,
            vmem_limit_bytes=64 * 1024 * 1024,
        ),
    )(x, w_mat)


# device time: 713343 ns/iter; 1.9784x vs baseline; 1.1597x over previous
import jax
import jax.numpy as jnp
from jax import lax
from jax.experimental import pallas as pl
from jax.experimental.pallas import tpu as pltpu

N_DEV = 4
M = 4096
KS = 1024
N = 8192
NH = N // 2
MC = M // N_DEV
NT = 1024
H_TILES = NH // NT


def kernel(x, w_mat):
    def body(x_hbm, w_hbm, out_hbm, stage, xcbf, wtbf, buf_a, buf_b,
             pbuf, local_sem, send_r, recv_r, send_l, recv_l):
        i = lax.axis_index("i")
        right = lax.rem(i + 1, N_DEV)
        left = lax.rem(i + N_DEV - 1, N_DEV)

        bar = pltpu.get_barrier_semaphore()

        def barrier():
            for nbr in (left, right):
                pl.semaphore_signal(
                    bar, inc=1, device_id=(nbr,),
                    device_id_type=pl.DeviceIdType.MESH)
            pl.semaphore_wait(bar, 2)

        def load_x_chunk(c):
            cp = pltpu.make_async_copy(
                x_hbm.at[pl.ds(c * MC, MC), :], stage, local_sem)
            cp.start()
            cp.wait()
            xcbf[...] = stage[...].astype(jnp.bfloat16)

        def load_w_tile(col):
            cp = pltpu.make_async_copy(
                w_hbm.at[:, pl.ds(col, NT)], stage, local_sem)
            cp.start()
            cp.wait()
            wtbf[...] = stage[...].astype(jnp.bfloat16)

        def gemm_half(c, dst, col0):
            load_x_chunk(c)

            def tile(nt, _):
                col = col0 + nt * NT
                load_w_tile(col)
                t = jnp.dot(xcbf[...], wtbf[...],
                            preferred_element_type=jnp.float32)
                dst[:, pl.ds(col, NT)] = t.astype(jnp.bfloat16)
                return 0

            lax.fori_loop(0, H_TILES, tile, 0)

        def gemm_full(c, dst):
            load_x_chunk(c)

            def tile(nt, _):
                load_w_tile(nt * NT)
                t = jnp.dot(xcbf[...], wtbf[...],
                            preferred_element_type=jnp.float32)
                dst[:, pl.ds(nt * NT, NT)] = t.astype(jnp.bfloat16)
                return 0

            lax.fori_loop(0, 2 * H_TILES, tile, 0)

        def combine(dst, other, relu=False):
            def tile(nt, _):
                sl = pl.ds(nt * NT, NT)
                t = dst[:, sl].astype(jnp.float32) \
                    + other[:, sl].astype(jnp.float32)
                if relu:
                    t = jnp.maximum(t, 0.0)
                dst[:, sl] = t.astype(jnp.bfloat16)
                return 0

            lax.fori_loop(0, 2 * H_TILES, tile, 0)

        def ring_start(src, dst):
            r = pltpu.make_async_remote_copy(
                src_ref=src.at[:, pl.ds(0, NH)],
                dst_ref=dst.at[:, pl.ds(0, NH)],
                send_sem=send_r, recv_sem=recv_r,
                device_id=(right,), device_id_type=pl.DeviceIdType.MESH)
            l = pltpu.make_async_remote_copy(
                src_ref=src.at[:, pl.ds(NH, NH)],
                dst_ref=dst.at[:, pl.ds(NH, NH)],
                send_sem=send_l, recv_sem=recv_l,
                device_id=(left,), device_id_type=pl.DeviceIdType.MESH)
            r.start()
            l.start()
            return r, l

        def ring_wait(rl):
            r, l = rl
            r.wait()
            l.wait()

        def store_half(buf, c, col0):
            def tile(nt, _):
                sl = pl.ds(col0 + nt * NT, NT)
                stage[...] = buf[:, sl].astype(jnp.float32)
                cp = pltpu.make_async_copy(
                    stage, out_hbm.at[pl.ds(c * MC, MC), sl], local_sem)
                cp.start()
                cp.wait()
                return 0

            lax.fori_loop(0, H_TILES, tile, 0)

        barrier()

        gemm_full(i, buf_a)
        barrier()
        rs0 = ring_start(buf_a, buf_b)
        gemm_half(lax.rem(i + 3, N_DEV), pbuf, 0)
        gemm_half(lax.rem(i + 1, N_DEV), pbuf, NH)
        ring_wait(rs0)
        combine(buf_b, pbuf)

        barrier()
        rs1 = ring_start(buf_b, buf_a)
        gemm_full(lax.rem(i + 2, N_DEV), pbuf)
        ring_wait(rs1)
        combine(buf_a, pbuf)

        barrier()
        rs2 = ring_start(buf_a, buf_b)
        gemm_half(lax.rem(i + 1, N_DEV), pbuf, 0)
        gemm_half(lax.rem(i + 3, N_DEV), pbuf, NH)
        ring_wait(rs2)
        combine(buf_b, pbuf, relu=True)

        barrier()
        ag0 = ring_start(buf_b, buf_a)
        store_half(buf_b, lax.rem(i + 1, N_DEV), 0)
        store_half(buf_b, lax.rem(i + 3, N_DEV), NH)
        ring_wait(ag0)

        barrier()
        ag1 = ring_start(buf_a, buf_b)
        store_half(buf_a, i, 0)
        store_half(buf_a, i, NH)
        ring_wait(ag1)

        barrier()
        ag2 = ring_start(buf_b, buf_a)
        store_half(buf_b, lax.rem(i + 3, N_DEV), 0)
        store_half(buf_b, lax.rem(i + 1, N_DEV), NH)
        ring_wait(ag2)

        store_half(buf_a, lax.rem(i + 2, N_DEV), 0)
        store_half(buf_a, lax.rem(i + 2, N_DEV), NH)

        barrier()

    return pl.pallas_call(
        body,
        out_shape=jax.ShapeDtypeStruct((M, N), jnp.float32),
        in_specs=[
            pl.BlockSpec(memory_space=pl.ANY),
            pl.BlockSpec(memory_space=pl.ANY),
        ],
        out_specs=pl.BlockSpec(memory_space=pl.ANY),
        scratch_shapes=[
            pltpu.VMEM((MC, NT), jnp.float32),
            pltpu.VMEM((MC, KS), jnp.bfloat16),
            pltpu.VMEM((KS, NT), jnp.bfloat16),
            pltpu.VMEM((MC, N), jnp.bfloat16),
            pltpu.VMEM((MC, N), jnp.bfloat16),
            pltpu.VMEM((MC, N), jnp.bfloat16),
            pltpu.SemaphoreType.DMA,
            pltpu.SemaphoreType.DMA,
            pltpu.SemaphoreType.DMA,
            pltpu.SemaphoreType.DMA,
            pltpu.SemaphoreType.DMA,
        ],
        compiler_params=pltpu.CompilerParams(
            collective_id=0,
            vmem_limit_bytes=64 * 1024 * 1024,
        ),
    )(x, w_mat)


# device time: 684664 ns/iter; 2.0612x vs baseline; 1.0419x over previous
import jax
import jax.numpy as jnp
from jax import lax
from jax.experimental import pallas as pl
from jax.experimental.pallas import tpu as pltpu

N_DEV = 4
M = 4096
KS = 1024
N = 8192
NH = N // 2
SUB = NH // 2
MC = M // N_DEV
NT = 1024
H_TILES = NH // NT
S_TILES = SUB // NT


def kernel(x, w_mat):
    def body(x_hbm, w_hbm, out_hbm, stage, xcbf, wtbf, buf_a, buf_b,
             pbuf, local_sem, send_r, recv_r, send_l, recv_l):
        i = lax.axis_index("i")
        right = lax.rem(i + 1, N_DEV)
        left = lax.rem(i + N_DEV - 1, N_DEV)

        bar = pltpu.get_barrier_semaphore()

        def barrier():
            for nbr in (left, right):
                pl.semaphore_signal(
                    bar, inc=1, device_id=(nbr,),
                    device_id_type=pl.DeviceIdType.MESH)
            pl.semaphore_wait(bar, 2)

        def load_x_chunk(c):
            cp = pltpu.make_async_copy(
                x_hbm.at[pl.ds(c * MC, MC), :], stage, local_sem)
            cp.start()
            cp.wait()
            xcbf[...] = stage[...].astype(jnp.bfloat16)

        def load_w_tile(col):
            cp = pltpu.make_async_copy(
                w_hbm.at[:, pl.ds(col, NT)], stage, local_sem)
            cp.start()
            cp.wait()
            wtbf[...] = stage[...].astype(jnp.bfloat16)

        def gemm_cols(col0, dst, n_tiles):
            def tile(nt, _):
                col = col0 + nt * NT
                load_w_tile(col)
                t = jnp.dot(xcbf[...], wtbf[...],
                            preferred_element_type=jnp.float32)
                dst[:, pl.ds(col, NT)] = t.astype(jnp.bfloat16)
                return 0

            lax.fori_loop(0, n_tiles, tile, 0)

        def gemm_half(c, dst, col0):
            load_x_chunk(c)
            gemm_cols(col0, dst, H_TILES)

        def gemm_full(c, dst):
            load_x_chunk(c)
            gemm_cols(0, dst, 2 * H_TILES)

        def combine(dst, other, relu=False):
            def tile(nt, _):
                sl = pl.ds(nt * NT, NT)
                t = dst[:, sl].astype(jnp.float32) \
                    + other[:, sl].astype(jnp.float32)
                if relu:
                    t = jnp.maximum(t, 0.0)
                dst[:, sl] = t.astype(jnp.bfloat16)
                return 0

            lax.fori_loop(0, 2 * H_TILES, tile, 0)

        def sub_start(src, dst, k):
            r = pltpu.make_async_remote_copy(
                src_ref=src.at[:, pl.ds(k * SUB, SUB)],
                dst_ref=dst.at[:, pl.ds(k * SUB, SUB)],
                send_sem=send_r.at[k], recv_sem=recv_r.at[k],
                device_id=(right,), device_id_type=pl.DeviceIdType.MESH)
            l = pltpu.make_async_remote_copy(
                src_ref=src.at[:, pl.ds(NH + k * SUB, SUB)],
                dst_ref=dst.at[:, pl.ds(NH + k * SUB, SUB)],
                send_sem=send_l.at[k], recv_sem=recv_l.at[k],
                device_id=(left,), device_id_type=pl.DeviceIdType.MESH)
            r.start()
            l.start()
            return r, l

        def ring_start(src, dst):
            r = pltpu.make_async_remote_copy(
                src_ref=src.at[:, pl.ds(0, NH)],
                dst_ref=dst.at[:, pl.ds(0, NH)],
                send_sem=send_r.at[0], recv_sem=recv_r.at[0],
                device_id=(right,), device_id_type=pl.DeviceIdType.MESH)
            l = pltpu.make_async_remote_copy(
                src_ref=src.at[:, pl.ds(NH, NH)],
                dst_ref=dst.at[:, pl.ds(NH, NH)],
                send_sem=send_l.at[0], recv_sem=recv_l.at[0],
                device_id=(left,), device_id_type=pl.DeviceIdType.MESH)
            r.start()
            l.start()
            return r, l

        def ring_wait(rl):
            r, l = rl
            r.wait()
            l.wait()

        def store_cols(buf, c, col0, n_tiles):
            def tile(nt, _):
                sl = pl.ds(col0 + nt * NT, NT)
                stage[...] = buf[:, sl].astype(jnp.float32)
                cp = pltpu.make_async_copy(
                    stage, out_hbm.at[pl.ds(c * MC, MC), sl], local_sem)
                cp.start()
                cp.wait()
                return 0

            lax.fori_loop(0, n_tiles, tile, 0)

        barrier()

        load_x_chunk(i)
        gemm_cols(0, buf_a, S_TILES)
        gemm_cols(NH, buf_a, S_TILES)
        rs0a = sub_start(buf_a, buf_b, 0)
        gemm_cols(SUB, buf_a, S_TILES)
        gemm_cols(NH + SUB, buf_a, S_TILES)
        rs0b = sub_start(buf_a, buf_b, 1)
        gemm_half(lax.rem(i + 3, N_DEV), pbuf, 0)
        gemm_half(lax.rem(i + 1, N_DEV), pbuf, NH)
        ring_wait(rs0a)
        ring_wait(rs0b)
        combine(buf_b, pbuf)

        barrier()
        rs1 = ring_start(buf_b, buf_a)
        gemm_full(lax.rem(i + 2, N_DEV), pbuf)
        ring_wait(rs1)
        combine(buf_a, pbuf)

        barrier()
        rs2 = ring_start(buf_a, buf_b)
        gemm_half(lax.rem(i + 1, N_DEV), pbuf, 0)
        gemm_half(lax.rem(i + 3, N_DEV), pbuf, NH)
        ring_wait(rs2)
        combine(buf_b, pbuf, relu=True)

        barrier()
        ag0 = ring_start(buf_b, buf_a)
        store_cols(buf_b, lax.rem(i + 1, N_DEV), 0, H_TILES)
        store_cols(buf_b, lax.rem(i + 3, N_DEV), NH, H_TILES)
        ring_wait(ag0)

        barrier()
        ag1 = ring_start(buf_a, buf_b)
        store_cols(buf_a, i, 0, H_TILES)
        store_cols(buf_a, i, NH, H_TILES)
        ring_wait(ag1)

        barrier()
        ag2a = sub_start(buf_b, buf_a, 0)
        ag2b = sub_start(buf_b, buf_a, 1)
        store_cols(buf_b, lax.rem(i + 3, N_DEV), 0, H_TILES)
        store_cols(buf_b, lax.rem(i + 1, N_DEV), NH, H_TILES)
        ring_wait(ag2a)
        o2 = lax.rem(i + 2, N_DEV)
        store_cols(buf_a, o2, 0, S_TILES)
        store_cols(buf_a, o2, NH, S_TILES)
        ring_wait(ag2b)
        store_cols(buf_a, o2, SUB, S_TILES)
        store_cols(buf_a, o2, NH + SUB, S_TILES)

        barrier()

    return pl.pallas_call(
        body,
        out_shape=jax.ShapeDtypeStruct((M, N), jnp.float32),
        in_specs=[
            pl.BlockSpec(memory_space=pl.ANY),
            pl.BlockSpec(memory_space=pl.ANY),
        ],
        out_specs=pl.BlockSpec(memory_space=pl.ANY),
        scratch_shapes=[
            pltpu.VMEM((MC, NT), jnp.float32),
            pltpu.VMEM((MC, KS), jnp.bfloat16),
            pltpu.VMEM((KS, NT), jnp.bfloat16),
            pltpu.VMEM((MC, N), jnp.bfloat16),
            pltpu.VMEM((MC, N), jnp.bfloat16),
            pltpu.VMEM((MC, N), jnp.bfloat16),
            pltpu.SemaphoreType.DMA,
            pltpu.SemaphoreType.DMA((2,)),
            pltpu.SemaphoreType.DMA((2,)),
            pltpu.SemaphoreType.DMA((2,)),
            pltpu.SemaphoreType.DMA((2,)),
        ],
        compiler_params=pltpu.CompilerParams(
            collective_id=0,
            vmem_limit_bytes=64 * 1024 * 1024,
        ),
    )(x, w_mat)


# device time: 677642 ns/iter; 2.0826x vs baseline; 1.0104x over previous
import jax
import jax.numpy as jnp
from jax import lax
from jax.experimental import pallas as pl
from jax.experimental.pallas import tpu as pltpu

N_DEV = 4
M = 4096
KS = 1024
N = 8192
NH = N // 2
SUB = NH // 2
MC = M // N_DEV
NT = 1024
H_TILES = NH // NT
S_TILES = SUB // NT


def kernel(x, w_mat):
    def body(x_hbm, w_hbm, out_hbm, stage, xcbf, wtbf, buf_a, buf_b,
             pbuf, local_sem, send_r, recv_r, send_l, recv_l):
        i = lax.axis_index("i")
        right = lax.rem(i + 1, N_DEV)
        left = lax.rem(i + N_DEV - 1, N_DEV)

        bar = pltpu.get_barrier_semaphore()

        def barrier():
            for nbr in (left, right):
                pl.semaphore_signal(
                    bar, inc=1, device_id=(nbr,),
                    device_id_type=pl.DeviceIdType.MESH)
            pl.semaphore_wait(bar, 2)

        def load_x_chunk(c):
            cp = pltpu.make_async_copy(
                x_hbm.at[pl.ds(c * MC, MC), :], stage, local_sem)
            cp.start()
            cp.wait()
            xcbf[...] = stage[...].astype(jnp.bfloat16)

        def load_w_tile(col):
            cp = pltpu.make_async_copy(
                w_hbm.at[:, pl.ds(col, NT)], stage, local_sem)
            cp.start()
            cp.wait()
            wtbf[...] = stage[...].astype(jnp.bfloat16)

        def gemm_cols(col0, dst, n_tiles):
            def tile(nt, _):
                col = col0 + nt * NT
                load_w_tile(col)
                t = jnp.dot(xcbf[...], wtbf[...],
                            preferred_element_type=jnp.float32)
                dst[:, pl.ds(col, NT)] = t.astype(jnp.bfloat16)
                return 0

            lax.fori_loop(0, n_tiles, tile, 0)

        def gemm_half(c, dst, col0):
            load_x_chunk(c)
            gemm_cols(col0, dst, H_TILES)

        def gemm_full(c, dst):
            load_x_chunk(c)
            gemm_cols(0, dst, 2 * H_TILES)

        def combine(dst, other, relu=False):
            def tile(nt, _):
                sl = pl.ds(nt * NT, NT)
                t = dst[:, sl].astype(jnp.float32) \
                    + other[:, sl].astype(jnp.float32)
                if relu:
                    t = jnp.maximum(t, 0.0)
                dst[:, sl] = t.astype(jnp.bfloat16)
                return 0

            lax.fori_loop(0, 2 * H_TILES, tile, 0)

        def combine_sub(dst, other, k, relu=False):
            def tile_at(base):
                def tile(nt, _):
                    sl = pl.ds(base + nt * NT, NT)
                    t = dst[:, sl].astype(jnp.float32) \
                        + other[:, sl].astype(jnp.float32)
                    if relu:
                        t = jnp.maximum(t, 0.0)
                    dst[:, sl] = t.astype(jnp.bfloat16)
                    return 0
                return tile

            lax.fori_loop(0, S_TILES, tile_at(k * SUB), 0)
            lax.fori_loop(0, S_TILES, tile_at(NH + k * SUB), 0)

        def sub_start(src, dst, k):
            r = pltpu.make_async_remote_copy(
                src_ref=src.at[:, pl.ds(k * SUB, SUB)],
                dst_ref=dst.at[:, pl.ds(k * SUB, SUB)],
                send_sem=send_r.at[k], recv_sem=recv_r.at[k],
                device_id=(right,), device_id_type=pl.DeviceIdType.MESH)
            l = pltpu.make_async_remote_copy(
                src_ref=src.at[:, pl.ds(NH + k * SUB, SUB)],
                dst_ref=dst.at[:, pl.ds(NH + k * SUB, SUB)],
                send_sem=send_l.at[k], recv_sem=recv_l.at[k],
                device_id=(left,), device_id_type=pl.DeviceIdType.MESH)
            r.start()
            l.start()
            return r, l

        def ring_start(src, dst):
            r = pltpu.make_async_remote_copy(
                src_ref=src.at[:, pl.ds(0, NH)],
                dst_ref=dst.at[:, pl.ds(0, NH)],
                send_sem=send_r.at[0], recv_sem=recv_r.at[0],
                device_id=(right,), device_id_type=pl.DeviceIdType.MESH)
            l = pltpu.make_async_remote_copy(
                src_ref=src.at[:, pl.ds(NH, NH)],
                dst_ref=dst.at[:, pl.ds(NH, NH)],
                send_sem=send_l.at[0], recv_sem=recv_l.at[0],
                device_id=(left,), device_id_type=pl.DeviceIdType.MESH)
            r.start()
            l.start()
            return r, l

        def ring_wait(rl):
            r, l = rl
            r.wait()
            l.wait()

        def store_cols(buf, c, col0, n_tiles):
            def tile(nt, _):
                sl = pl.ds(col0 + nt * NT, NT)
                stage[...] = buf[:, sl].astype(jnp.float32)
                cp = pltpu.make_async_copy(
                    stage, out_hbm.at[pl.ds(c * MC, MC), sl], local_sem)
                cp.start()
                cp.wait()
                return 0

            lax.fori_loop(0, n_tiles, tile, 0)

        barrier()

        load_x_chunk(i)
        gemm_cols(0, buf_a, S_TILES)
        gemm_cols(NH, buf_a, S_TILES)
        rs0a = sub_start(buf_a, buf_b, 0)
        gemm_cols(SUB, buf_a, S_TILES)
        gemm_cols(NH + SUB, buf_a, S_TILES)
        rs0b = sub_start(buf_a, buf_b, 1)
        gemm_half(lax.rem(i + 3, N_DEV), pbuf, 0)
        gemm_half(lax.rem(i + 1, N_DEV), pbuf, NH)
        ring_wait(rs0a)
        ring_wait(rs0b)
        combine(buf_b, pbuf)

        barrier()
        rs1a = sub_start(buf_b, buf_a, 0)
        rs1b = sub_start(buf_b, buf_a, 1)
        gemm_full(lax.rem(i + 2, N_DEV), pbuf)
        ring_wait(rs1a)
        combine_sub(buf_a, pbuf, 0)
        ring_wait(rs1b)
        combine_sub(buf_a, pbuf, 1)

        barrier()
        rs2a = sub_start(buf_a, buf_b, 0)
        rs2b = sub_start(buf_a, buf_b, 1)
        gemm_half(lax.rem(i + 1, N_DEV), pbuf, 0)
        gemm_half(lax.rem(i + 3, N_DEV), pbuf, NH)
        ring_wait(rs2a)
        combine_sub(buf_b, pbuf, 0, relu=True)
        ring_wait(rs2b)

        barrier()
        ag0a = sub_start(buf_b, buf_a, 0)
        combine_sub(buf_b, pbuf, 1, relu=True)
        ag0b = sub_start(buf_b, buf_a, 1)
        store_cols(buf_b, lax.rem(i + 1, N_DEV), 0, H_TILES)
        store_cols(buf_b, lax.rem(i + 3, N_DEV), NH, H_TILES)
        ring_wait(ag0a)
        ring_wait(ag0b)

        barrier()
        ag1 = ring_start(buf_a, buf_b)
        store_cols(buf_a, i, 0, H_TILES)
        store_cols(buf_a, i, NH, H_TILES)
        ring_wait(ag1)

        barrier()
        ag2a = sub_start(buf_b, buf_a, 0)
        ag2b = sub_start(buf_b, buf_a, 1)
        store_cols(buf_b, lax.rem(i + 3, N_DEV), 0, H_TILES)
        store_cols(buf_b, lax.rem(i + 1, N_DEV), NH, H_TILES)
        ring_wait(ag2a)
        o2 = lax.rem(i + 2, N_DEV)
        store_cols(buf_a, o2, 0, S_TILES)
        store_cols(buf_a, o2, NH, S_TILES)
        ring_wait(ag2b)
        store_cols(buf_a, o2, SUB, S_TILES)
        store_cols(buf_a, o2, NH + SUB, S_TILES)

        barrier()

    return pl.pallas_call(
        body,
        out_shape=jax.ShapeDtypeStruct((M, N), jnp.float32),
        in_specs=[
            pl.BlockSpec(memory_space=pl.ANY),
            pl.BlockSpec(memory_space=pl.ANY),
        ],
        out_specs=pl.BlockSpec(memory_space=pl.ANY),
        scratch_shapes=[
            pltpu.VMEM((MC, NT), jnp.float32),
            pltpu.VMEM((MC, KS), jnp.bfloat16),
            pltpu.VMEM((KS, NT), jnp.bfloat16),
            pltpu.VMEM((MC, N), jnp.bfloat16),
            pltpu.VMEM((MC, N), jnp.bfloat16),
            pltpu.VMEM((MC, N), jnp.bfloat16),
            pltpu.SemaphoreType.DMA,
            pltpu.SemaphoreType.DMA((2,)),
            pltpu.SemaphoreType.DMA((2,)),
            pltpu.SemaphoreType.DMA((2,)),
            pltpu.SemaphoreType.DMA((2,)),
        ],
        compiler_params=pltpu.CompilerParams(
            collective_id=0,
            vmem_limit_bytes=64 * 1024 * 1024,
        ),
    )(x, w_mat)
